# baseline (speedup 1.0000x reference)
import functools

import jax
import jax.numpy as jnp
from jax import lax
from jax.experimental import pallas as pl
from jax.experimental.pallas import tpu as pltpu, tpu_sc as plsc

N = 10000
NPAD = 10240
D = 128
H = 64
E = 320000
NC, NS = 2, 16
NW = NC * NS
CH = 128
NCHUNK = 80
EW = E // NW
NBUF = 4
NPAIR = 2048
PPW = NPAIR // NW
STRIPE = NPAD // NS

_MESH = plsc.VectorSubcoreMesh(core_axis_name="c", subcore_axis_name="s")
_SC_PARAMS = pltpu.CompilerParams(
    needs_layout_passes=False, use_tc_tiling_on_sc=False
)


def _wid():
    return lax.axis_index("s") * NC + lax.axis_index("c")


@functools.partial(
    pl.kernel,
    out_type=jax.ShapeDtypeStruct((NW, NPAD), jnp.float32),
    mesh=_MESH,
    compiler_params=_SC_PARAMS,
    scratch_types=[
        pltpu.VMEM((EW,), jnp.int32),
        pltpu.VMEM((NPAD,), jnp.float32),
    ],
)
def _deg(ei_hbm, out_hbm, dstv, degv):
    wid = _wid()
    pltpu.sync_copy(ei_hbm.at[1, pl.ds(wid * EW, EW)], dstv)
    zeros16 = jnp.zeros((16,), jnp.float32)
    ones16 = jnp.ones((16,), jnp.float32)

    def zb(i, _):
        degv[pl.ds(i * 16, 16)] = zeros16
        return 0

    lax.fori_loop(0, NPAD // 16, zb, 0, unroll=4)

    def eb(i, _):
        idx = dstv[pl.ds(i * 16, 16)]
        plsc.addupdate_scatter(degv, [idx], ones16)
        return 0

    lax.fori_loop(0, EW // 16, eb, 0, unroll=4)
    pltpu.sync_copy(degv, out_hbm.at[wid])


def _h_body(x_ref, w_ref, degp_ref, out_ref):
    deg = 1.0 + jnp.sum(degp_ref[...], axis=0)
    dinv = lax.rsqrt(deg)
    h = jnp.dot(x_ref[...], w_ref[...], preferred_element_type=jnp.float32)
    out_ref[...] = h * dinv[:, None]


def _h(x_pad, w, degp):
    blk = NPAD // 8
    return pl.pallas_call(
        _h_body,
        grid=(8,),
        in_specs=[
            pl.BlockSpec((blk, D), lambda i: (i, 0)),
            pl.BlockSpec((D, H), lambda i: (0, 0)),
            pl.BlockSpec((NW, blk), lambda i: (0, i)),
        ],
        out_specs=pl.BlockSpec((blk, H), lambda i: (i, 0)),
        out_shape=jax.ShapeDtypeStruct((NPAD, H), jnp.float32),
    )(x_pad, w, degp)


@functools.partial(
    pl.kernel,
    out_type=jax.ShapeDtypeStruct((NC, NPAD, H), jnp.float32),
    mesh=_MESH,
    compiler_params=_SC_PARAMS,
    scratch_types=[
        pltpu.VMEM((EW + 2 * CH,), jnp.int32),
        pltpu.VMEM((EW + 2 * CH,), jnp.int32),
        pltpu.VMEM((NCHUNK, CH), jnp.int32),
        pltpu.VMEM((2 * NPAIR,), jnp.int32),
        pltpu.VMEM((NPAD,), jnp.float32),
        pltpu.VMEM((NBUF, CH, H), jnp.float32),
        pltpu.VMEM_SHARED((NPAD, H), jnp.float32),
        pltpu.SemaphoreType.DMA((NBUF,)),
        pltpu.SemaphoreType.DMA((NBUF,)),
        pltpu.SemaphoreType.DMA,
    ],
)
def _msg(ei_hbm, pairs_hbm, hp_hbm, out_hbm, srcf, dstf,
         dstv, idxp, flagv, rows, acc, gsem, ssem, isem):
    cid = lax.axis_index("c")
    sid = lax.axis_index("s")
    wid = sid * NC + cid
    pltpu.sync_copy(ei_hbm.at[0, pl.ds(wid * EW, EW)], srcf.at[pl.ds(0, EW)])
    pltpu.sync_copy(ei_hbm.at[1, pl.ds(wid * EW, EW)], dstf.at[pl.ds(0, EW)])
    pltpu.sync_copy(pairs_hbm, idxp)

    @pl.when(cid == 0)
    def _():
        pltpu.async_copy(
            hp_hbm.at[pl.ds(sid * STRIPE, STRIPE)],
            acc.at[pl.ds(sid * STRIPE, STRIPE)],
            isem,
        )

    @pl.when(cid != 0)
    def _():
        zeros16 = jnp.zeros((16,), jnp.float32)

        def zb(i, _):
            rows[0, i, pl.ds(0, 16)] = zeros16
            rows[0, i, pl.ds(16, 16)] = zeros16
            rows[0, i, pl.ds(32, 16)] = zeros16
            rows[0, i, pl.ds(48, 16)] = zeros16
            return 0

        lax.fori_loop(0, CH, zb, 0, unroll=4)
        for t in range(STRIPE // CH):
            pltpu.async_copy(
                rows.at[0], acc.at[pl.ds(sid * STRIPE + t * CH, CH)], isem
            )

    zeros16 = jnp.zeros((16,), jnp.float32)
    ones16 = jnp.ones((16,), jnp.float32)

    def fz(i, _):
        flagv[pl.ds(i * 16, 16)] = zeros16
        return 0

    lax.fori_loop(0, NPAD // 16, fz, 0, unroll=4)

    def fs(i, _):
        ids = idxp[pl.ds(i * 16, 16)]
        plsc.store_scatter(flagv, [ids], ones16)
        return 0

    lax.fori_loop(0, 2 * NPAIR // 16, fs, 0, unroll=4)

    def fb(i, off):
        sv = srcf[pl.ds(i * 16, 16)]
        dv = dstf[pl.ds(i * 16, 16)]
        fl = plsc.load_gather(flagv, [dv])
        m = fl > 0.0
        plsc.store_compressed(srcf.at[pl.ds(off, 16)], sv, mask=m)
        plsc.store_compressed(dstf.at[pl.ds(off, 16)], dv, mask=m)
        return off + plsc.all_reduce_population_count(m)[0]

    off = lax.fori_loop(0, EW // 16, fb, jnp.int32(0), unroll=2)

    lanes16 = lax.broadcasted_iota(jnp.int32, (16,), 0)
    for k in range(NBUF * 2):
        srcf[pl.ds(off + k * 16, 16)] = lanes16 + k * 16
        dstf[pl.ds(off + k * 16, 16)] = N + lanes16 + k * 16
    nc = (off + CH - 1) // CH

    def cpy(g, _):
        for u in range(CH // 16):
            dstv[g, pl.ds(u * 16, 16)] = dstf[pl.ds(g * CH + u * 16, 16)]
        return 0

    lax.fori_loop(0, nc, cpy, 0)

    @pl.when(cid == 0)
    def _():
        pltpu.make_async_copy(
            hp_hbm.at[pl.ds(sid * STRIPE, STRIPE)],
            acc.at[pl.ds(sid * STRIPE, STRIPE)],
            isem,
        ).wait()

    @pl.when(cid != 0)
    def _():
        for t in range(STRIPE // CH):
            pltpu.make_async_copy(
                rows.at[0], acc.at[pl.ds(sid * STRIPE + t * CH, CH)], isem
            ).wait()

    plsc.subcore_barrier()

    def _sg(b, j):
        pltpu.async_copy(hp_hbm.at[srcf.at[pl.ds(j * CH, CH)]], rows.at[b], gsem.at[b])

    def _wg(b, j):
        pltpu.make_async_copy(
            hp_hbm.at[srcf.at[pl.ds(j * CH, CH)]], rows.at[b], gsem.at[b]
        ).wait()

    def _ss(b, j):
        pltpu.async_copy(rows.at[b], acc.at[dstv.at[j]], ssem.at[b], add=True)

    def _ws(b, j):
        pltpu.make_async_copy(rows.at[b], acc.at[dstv.at[j]], ssem.at[b]).wait()

    for b in range(NBUF):

        @pl.when(b < nc)
        def _():
            _sg(b, b)

    def body(g, _):
        for b in range(NBUF):
            j = g * NBUF + b

            @pl.when(j < nc)
            def _():
                _wg(b, j)
                _ss(b, j)
                _ws(b, j)

                @pl.when(j + NBUF < nc)
                def _():
                    _sg(b, j + NBUF)

        return 0

    lax.fori_loop(0, (nc + NBUF - 1) // NBUF, body, 0)

    plsc.subcore_barrier()
    pltpu.sync_copy(
        acc.at[pl.ds(sid * STRIPE, STRIPE)],
        out_hbm.at[cid, pl.ds(sid * STRIPE, STRIPE)],
    )


def _pq_body(part_ref, degp_ref, bconv_ref, wcat_ref, out_ref):
    deg = 1.0 + jnp.sum(degp_ref[...], axis=0)
    dinv = lax.rsqrt(deg)
    tot = part_ref[0] + part_ref[1]
    pre = tot * dinv[:, None] + bconv_ref[...]
    act = jnp.maximum(pre, 0.01 * pre)
    out_ref[...] = jnp.dot(act, wcat_ref[...], preferred_element_type=jnp.float32)


def _pq(part, degp, bconv, wcat):
    blk = NPAD // 8
    return pl.pallas_call(
        _pq_body,
        grid=(8,),
        in_specs=[
            pl.BlockSpec((NC, blk, H), lambda i: (0, i, 0)),
            pl.BlockSpec((NW, blk), lambda i: (0, i)),
            pl.BlockSpec((1, H), lambda i: (0, 0)),
            pl.BlockSpec((H, 32), lambda i: (0, 0)),
        ],
        out_specs=pl.BlockSpec((blk, 32), lambda i: (i, 0)),
        out_shape=jax.ShapeDtypeStruct((NPAD, 32), jnp.float32),
    )(part, degp, bconv, wcat)


@functools.partial(
    pl.kernel,
    out_type=jax.ShapeDtypeStruct((NPAIR,), jnp.float32),
    mesh=_MESH,
    compiler_params=_SC_PARAMS,
    scratch_types=[
        pltpu.VMEM((2 * PPW,), jnp.int32),
        pltpu.VMEM((PPW,), jnp.int32),
        pltpu.VMEM((PPW,), jnp.int32),
        pltpu.VMEM((PPW, 32), jnp.float32),
        pltpu.VMEM((PPW, 32), jnp.float32),
        pltpu.VMEM((48,), jnp.float32),
        pltpu.VMEM((PPW,), jnp.float32),
        pltpu.SemaphoreType.DMA,
    ],
)
def _pair(pq_hbm, pairs_hbm, c_hbm, out_hbm, pairv, iv, jv, pi, qj, cv, outv, sem):
    wid = _wid()
    pltpu.sync_copy(pairs_hbm.at[pl.ds(wid * 2 * PPW, 2 * PPW)], pairv)
    pltpu.sync_copy(c_hbm, cv)
    lanes2 = 2 * lax.broadcasted_iota(jnp.int32, (16,), 0)
    for g in range(PPW // 16):
        iv[pl.ds(g * 16, 16)] = plsc.load_gather(pairv, [lanes2 + g * 32])
        jv[pl.ds(g * 16, 16)] = plsc.load_gather(pairv, [lanes2 + g * 32 + 1])
    pltpu.async_copy(pq_hbm.at[iv], pi, sem).wait()
    pltpu.async_copy(pq_hbm.at[jv], qj, sem).wait()
    blin = cv[pl.ds(0, 16)]
    w1 = cv[pl.ds(16, 16)]
    b1 = cv[pl.ds(32, 16)]
    lanes = lax.broadcasted_iota(jnp.int32, (16,), 0)
    for g in range(PPW // 16):
        accv = jnp.zeros((16,), jnp.float32)
        for k0 in range(16):
            k = g * 16 + k0
            v = pi[k, pl.ds(0, 16)] + qj[k, pl.ds(16, 16)] + blin
            v = jnp.maximum(v, 0.01 * v)
            s = jnp.sum(v * w1)
            accv = jnp.where(lanes == k0, s, accv)
        outv[pl.ds(g * 16, 16)] = 1.0 / (1.0 + jnp.exp(-(accv + b1)))
    pltpu.sync_copy(outv, out_hbm.at[pl.ds(wid * PPW, PPW)])


def kernel(x, edge_index, index, W_conv, b_conv, W_lin, b_lin, W_lin1, b_lin1):
    f32 = jnp.float32
    ei = edge_index.astype(jnp.int32)
    pairs = index.astype(jnp.int32).reshape(2 * NPAIR)
    x_pad = jnp.pad(x, ((0, NPAD - N), (0, 0)))

    degp = _deg(ei)
    hp = _h(x_pad, W_conv, degp)
    part = _msg(ei, pairs, hp)
    wcat = jnp.concatenate([W_lin[:H], W_lin[H:]], axis=1)
    pq = _pq(part, degp, b_conv.reshape(1, H), wcat)
    consts = jnp.concatenate([b_lin, W_lin1[:, 0], jnp.full((16,), b_lin1[0], f32)])
    outf = _pair(pq, pairs, consts)
    return outf.reshape(NPAIR, 1)


# speedup vs baseline: 1.0999x; 1.0999x over previous
import functools

import jax
import jax.numpy as jnp
from jax import lax
from jax.experimental import pallas as pl
from jax.experimental.pallas import tpu as pltpu, tpu_sc as plsc

N = 10000
NPAD = 10240
D = 128
H = 64
E = 320000
NC, NS = 2, 16
NW = NC * NS
CH = 128
NCHUNK = 80
EW = E // NW
NBUF = 4
NPAIR = 2048
PPW = NPAIR // NW
STRIPE = NPAD // NS

_MESH = plsc.VectorSubcoreMesh(core_axis_name="c", subcore_axis_name="s")
_SC_PARAMS = pltpu.CompilerParams(
    needs_layout_passes=False, use_tc_tiling_on_sc=False
)


def _wid():
    return lax.axis_index("s") * NC + lax.axis_index("c")


@functools.partial(
    pl.kernel,
    out_type=jax.ShapeDtypeStruct((NW, NPAD), jnp.float32),
    mesh=_MESH,
    compiler_params=_SC_PARAMS,
    scratch_types=[
        pltpu.VMEM((EW,), jnp.int32),
        pltpu.VMEM((NPAD,), jnp.float32),
    ],
)
def _deg(ei_hbm, out_hbm, dstv, degv):
    wid = _wid()
    pltpu.sync_copy(ei_hbm.at[1, pl.ds(wid * EW, EW)], dstv)
    zeros16 = jnp.zeros((16,), jnp.float32)
    ones16 = jnp.ones((16,), jnp.float32)

    def zb(i, _):
        degv[pl.ds(i * 16, 16)] = zeros16
        return 0

    lax.fori_loop(0, NPAD // 16, zb, 0, unroll=4)

    def eb(i, _):
        idx = dstv[pl.ds(i * 16, 16)]
        plsc.addupdate_scatter(degv, [idx], ones16)
        return 0

    lax.fori_loop(0, EW // 16, eb, 0, unroll=4)
    pltpu.sync_copy(degv, out_hbm.at[wid])


def _h_body(x_ref, w_ref, degp_ref, out_ref, dinv_ref):
    deg = 1.0 + jnp.sum(degp_ref[...], axis=0)
    dinv = lax.rsqrt(deg)
    h = jnp.dot(x_ref[...], w_ref[...], preferred_element_type=jnp.float32)
    out_ref[...] = h * dinv[:, None]
    dinv_ref[...] = dinv.reshape(1, 1, -1)


def _h(x_pad, w, degp):
    blk = NPAD // 8
    return pl.pallas_call(
        _h_body,
        grid=(8,),
        in_specs=[
            pl.BlockSpec((blk, D), lambda i: (i, 0)),
            pl.BlockSpec((D, H), lambda i: (0, 0)),
            pl.BlockSpec((NW, blk), lambda i: (0, i)),
        ],
        out_specs=[
            pl.BlockSpec((blk, H), lambda i: (i, 0)),
            pl.BlockSpec((1, 1, blk), lambda i: (i, 0, 0)),
        ],
        out_shape=[
            jax.ShapeDtypeStruct((NPAD, H), jnp.float32),
            jax.ShapeDtypeStruct((8, 1, blk), jnp.float32),
        ],
    )(x_pad, w, degp)


@functools.partial(
    pl.kernel,
    out_type=jax.ShapeDtypeStruct((NC, NPAD // 2, 2 * H), jnp.float32),
    mesh=_MESH,
    compiler_params=_SC_PARAMS,
    scratch_types=[
        pltpu.VMEM((EW + 2 * CH,), jnp.int32),
        pltpu.VMEM((EW + 2 * CH,), jnp.int32),
        pltpu.VMEM((NCHUNK, CH), jnp.int32),
        pltpu.VMEM((2 * NPAIR,), jnp.int32),
        pltpu.VMEM((NPAD,), jnp.float32),
        pltpu.VMEM((NBUF, CH, H), jnp.float32),
        pltpu.VMEM((STRIPE,), jnp.float32),
        pltpu.VMEM((CH // 2, 2 * H), jnp.float32),
        pltpu.VMEM_SHARED((NPAD, H), jnp.float32),
        pltpu.SemaphoreType.DMA((NBUF,)),
        pltpu.SemaphoreType.DMA((NBUF,)),
        pltpu.SemaphoreType.DMA,
    ],
)
def _msg(ei_hbm, pairs_hbm, hp_hbm, dinv_hbm, out_hbm, srcf, dstf,
         dstv, idxp, flagv, rows, dinvb, packb, acc, gsem, ssem, isem):
    cid = lax.axis_index("c")
    sid = lax.axis_index("s")
    wid = sid * NC + cid
    pltpu.sync_copy(ei_hbm.at[0, pl.ds(wid * EW, EW)], srcf.at[pl.ds(0, EW)])
    pltpu.sync_copy(ei_hbm.at[1, pl.ds(wid * EW, EW)], dstf.at[pl.ds(0, EW)])
    pltpu.sync_copy(pairs_hbm, idxp)
    pltpu.sync_copy(dinv_hbm.at[sid], dinvb)

    @pl.when(cid == 0)
    def _():
        pltpu.async_copy(
            hp_hbm.at[pl.ds(sid * STRIPE, STRIPE)],
            acc.at[pl.ds(sid * STRIPE, STRIPE)],
            isem,
        )

    @pl.when(cid != 0)
    def _():
        zeros16 = jnp.zeros((16,), jnp.float32)

        def zb(i, _):
            rows[0, i, pl.ds(0, 16)] = zeros16
            rows[0, i, pl.ds(16, 16)] = zeros16
            rows[0, i, pl.ds(32, 16)] = zeros16
            rows[0, i, pl.ds(48, 16)] = zeros16
            return 0

        lax.fori_loop(0, CH, zb, 0, unroll=4)
        for t in range(STRIPE // CH):
            pltpu.async_copy(
                rows.at[0], acc.at[pl.ds(sid * STRIPE + t * CH, CH)], isem
            )

    zeros16 = jnp.zeros((16,), jnp.float32)
    ones16 = jnp.ones((16,), jnp.float32)

    def fz(i, _):
        flagv[pl.ds(i * 16, 16)] = zeros16
        return 0

    lax.fori_loop(0, NPAD // 16, fz, 0, unroll=4)

    def fs(i, _):
        ids = idxp[pl.ds(i * 16, 16)]
        plsc.store_scatter(flagv, [ids], ones16)
        return 0

    lax.fori_loop(0, 2 * NPAIR // 16, fs, 0, unroll=4)

    def fb(i, off):
        sv = srcf[pl.ds(i * 16, 16)]
        dv = dstf[pl.ds(i * 16, 16)]
        fl = plsc.load_gather(flagv, [dv])
        m = fl > 0.0
        plsc.store_compressed(srcf.at[pl.ds(off, 16)], sv, mask=m)
        plsc.store_compressed(dstf.at[pl.ds(off, 16)], dv, mask=m)
        return off + plsc.all_reduce_population_count(m)[0]

    off = lax.fori_loop(0, EW // 16, fb, jnp.int32(0), unroll=2)

    lanes16 = lax.broadcasted_iota(jnp.int32, (16,), 0)
    for k in range(NBUF * 2):
        srcf[pl.ds(off + k * 16, 16)] = lanes16 + k * 16
        dstf[pl.ds(off + k * 16, 16)] = N + lanes16 + k * 16
    nc = (off + CH - 1) // CH

    def cpy(g, _):
        for u in range(CH // 16):
            dstv[g, pl.ds(u * 16, 16)] = dstf[pl.ds(g * CH + u * 16, 16)]
        return 0

    lax.fori_loop(0, nc, cpy, 0)

    @pl.when(cid == 0)
    def _():
        pltpu.make_async_copy(
            hp_hbm.at[pl.ds(sid * STRIPE, STRIPE)],
            acc.at[pl.ds(sid * STRIPE, STRIPE)],
            isem,
        ).wait()

    @pl.when(cid != 0)
    def _():
        for t in range(STRIPE // CH):
            pltpu.make_async_copy(
                rows.at[0], acc.at[pl.ds(sid * STRIPE + t * CH, CH)], isem
            ).wait()

    plsc.subcore_barrier()

    def _sg(b, j):
        pltpu.async_copy(hp_hbm.at[srcf.at[pl.ds(j * CH, CH)]], rows.at[b], gsem.at[b])

    def _wg(b, j):
        pltpu.make_async_copy(
            hp_hbm.at[srcf.at[pl.ds(j * CH, CH)]], rows.at[b], gsem.at[b]
        ).wait()

    def _ss(b, j):
        pltpu.async_copy(rows.at[b], acc.at[dstv.at[j]], ssem.at[b], add=True)

    def _ws(b, j):
        pltpu.make_async_copy(rows.at[b], acc.at[dstv.at[j]], ssem.at[b]).wait()

    for b in range(NBUF):

        @pl.when(b < nc)
        def _():
            _sg(b, b)

    def body(g, _):
        for b in range(NBUF):
            j = g * NBUF + b

            @pl.when(j < nc)
            def _():
                _wg(b, j)
                _ss(b, j)
                _ws(b, j)

                @pl.when(j + NBUF < nc)
                def _():
                    _sg(b, j + NBUF)

        return 0

    lax.fori_loop(0, (nc + NBUF - 1) // NBUF, body, 0)

    plsc.subcore_barrier()

    def wb(t, _):
        pltpu.sync_copy(acc.at[pl.ds(sid * STRIPE + t * CH, CH)], rows.at[0])
        for r2 in range(CH // 2):
            base = t * CH + 2 * r2
            dv0 = plsc.load_gather(dinvb, [jnp.full((16,), base, jnp.int32)])
            dv1 = plsc.load_gather(dinvb, [jnp.full((16,), base + 1, jnp.int32)])
            for u in range(H // 16):
                packb[r2, pl.ds(u * 16, 16)] = rows[0, 2 * r2, pl.ds(u * 16, 16)] * dv0
                packb[r2, pl.ds(H + u * 16, 16)] = (
                    rows[0, 2 * r2 + 1, pl.ds(u * 16, 16)] * dv1
                )
        pltpu.sync_copy(
            packb,
            out_hbm.at[cid, pl.ds(sid * (STRIPE // 2) + t * (CH // 2), CH // 2)],
        )
        return 0

    lax.fori_loop(0, STRIPE // CH, wb, 0)


def _pq_body(part_ref, bconv2_ref, wcat2_ref, out_ref):
    tot = part_ref[0] + part_ref[1]
    pre = tot + bconv2_ref[...]
    act = jnp.maximum(pre, 0.01 * pre)
    out_ref[...] = jnp.dot(act, wcat2_ref[...], preferred_element_type=jnp.float32)


def _pq(part2, bconv2, wcat2):
    blk = NPAD // 16
    return pl.pallas_call(
        _pq_body,
        grid=(8,),
        in_specs=[
            pl.BlockSpec((NC, blk, 2 * H), lambda i: (0, i, 0)),
            pl.BlockSpec((1, 2 * H), lambda i: (0, 0)),
            pl.BlockSpec((2 * H, H), lambda i: (0, 0)),
        ],
        out_specs=pl.BlockSpec((blk, H), lambda i: (i, 0)),
        out_shape=jax.ShapeDtypeStruct((NPAD // 2, H), jnp.float32),
    )(part2, bconv2, wcat2)


@functools.partial(
    pl.kernel,
    out_type=jax.ShapeDtypeStruct((NPAIR,), jnp.float32),
    mesh=_MESH,
    compiler_params=_SC_PARAMS,
    scratch_types=[
        pltpu.VMEM((2 * PPW,), jnp.int32),
        pltpu.VMEM((PPW,), jnp.int32),
        pltpu.VMEM((PPW,), jnp.int32),
        pltpu.VMEM((PPW, 32), jnp.float32),
        pltpu.VMEM((PPW, 32), jnp.float32),
        pltpu.VMEM((48,), jnp.float32),
        pltpu.VMEM((PPW,), jnp.float32),
        pltpu.SemaphoreType.DMA,
    ],
)
def _pair(pq_hbm, pairs_hbm, c_hbm, out_hbm, pairv, iv, jv, pi, qj, cv, outv, sem):
    wid = _wid()
    pltpu.sync_copy(pairs_hbm.at[pl.ds(wid * 2 * PPW, 2 * PPW)], pairv)
    pltpu.sync_copy(c_hbm, cv)
    lanes2 = 2 * lax.broadcasted_iota(jnp.int32, (16,), 0)
    for g in range(PPW // 16):
        iv[pl.ds(g * 16, 16)] = plsc.load_gather(pairv, [lanes2 + g * 32])
        jv[pl.ds(g * 16, 16)] = plsc.load_gather(pairv, [lanes2 + g * 32 + 1])
    pltpu.async_copy(pq_hbm.at[iv], pi, sem).wait()
    pltpu.async_copy(pq_hbm.at[jv], qj, sem).wait()
    blin = cv[pl.ds(0, 16)]
    w1 = cv[pl.ds(16, 16)]
    b1 = cv[pl.ds(32, 16)]
    lanes = lax.broadcasted_iota(jnp.int32, (16,), 0)
    for g in range(PPW // 16):
        accv = jnp.zeros((16,), jnp.float32)
        for k0 in range(16):
            k = g * 16 + k0
            v = pi[k, pl.ds(0, 16)] + qj[k, pl.ds(16, 16)] + blin
            v = jnp.maximum(v, 0.01 * v)
            s = jnp.sum(v * w1)
            accv = jnp.where(lanes == k0, s, accv)
        outv[pl.ds(g * 16, 16)] = 1.0 / (1.0 + jnp.exp(-(accv + b1)))
    pltpu.sync_copy(outv, out_hbm.at[pl.ds(wid * PPW, PPW)])


def kernel(x, edge_index, index, W_conv, b_conv, W_lin, b_lin, W_lin1, b_lin1):
    f32 = jnp.float32
    ei = edge_index.astype(jnp.int32)
    pairs = index.astype(jnp.int32).reshape(2 * NPAIR)
    x_pad = jnp.pad(x, ((0, NPAD - N), (0, 0)))

    degp = _deg(ei)
    hp, dinv8 = _h(x_pad, W_conv, degp)
    part2 = _msg(ei, pairs, hp, dinv8.reshape(NS, STRIPE))
    wcat = jnp.concatenate([W_lin[:H], W_lin[H:]], axis=1)
    wcat2 = jnp.zeros((2 * H, H), f32).at[:H, :32].set(wcat).at[H:, 32:].set(wcat)
    bconv2 = jnp.concatenate([b_conv, b_conv]).reshape(1, 2 * H)
    pq = _pq(part2, bconv2, wcat2)
    consts = jnp.concatenate([b_lin, W_lin1[:, 0], jnp.full((16,), b_lin1[0], f32)])
    outf = _pair(pq.reshape(NPAD, 32), pairs, consts)
    return outf.reshape(NPAIR, 1)
